# baseline (speedup 1.0000x reference)
import functools

import jax
import jax.numpy as jnp
from jax import lax
from jax.experimental import pallas as pl
from jax.experimental.pallas import tpu as pltpu
from jax.experimental.pallas import tpu_sc as plsc

VOCAB = 49408
NUM_POS = 77
EMBED = 768
BATCH = 1024
ROWS = BATCH * NUM_POS

NUM_CORES = 2
NUM_SUBCORES = 16
NUM_WORKERS = NUM_CORES * NUM_SUBCORES
ROWS_PER_W = ROWS // NUM_WORKERS
CHUNK = 88
NUM_CHUNKS = ROWS_PER_W // CHUNK
LANES = 16
VREGS_PER_ROW = EMBED // LANES

_mesh = plsc.VectorSubcoreMesh(core_axis_name="c", subcore_axis_name="s")


@functools.partial(
    pl.kernel,
    mesh=_mesh,
    out_type=jax.ShapeDtypeStruct((ROWS, EMBED), jnp.float32),
    scratch_types=[
        pltpu.VMEM((CHUNK,), jnp.int32),
        pltpu.VMEM((NUM_POS, EMBED), jnp.float32),
        pltpu.VMEM((CHUNK, EMBED), jnp.float32),
        pltpu.SemaphoreType.DMA,
        pltpu.SemaphoreType.DMA,
    ],
)
def _emb_kernel(tok_hbm, table_hbm, pos_hbm, out_hbm, idx_v, pos_v, buf_v,
                sem_g, sem_p):
    wid = lax.axis_index("s") * NUM_CORES + lax.axis_index("c")
    wrow0 = wid * ROWS_PER_W

    pltpu.async_copy(pos_hbm, pos_v, sem_p).wait()

    def chunk_body(c, _):
        row0 = wrow0 + c * CHUNK

        pltpu.sync_copy(tok_hbm.at[pl.ds(row0, CHUNK)], idx_v)
        pltpu.async_copy(table_hbm.at[idx_v], buf_v, sem_g).wait()

        def add_row(k, phase):
            for j in range(VREGS_PER_ROW):
                sl = pl.ds(j * LANES, LANES)
                buf_v[k, sl] = buf_v[k, sl] + pos_v[phase, sl]
            return jnp.where(phase == NUM_POS - 1, 0, phase + 1)

        lax.fori_loop(0, CHUNK, add_row, lax.rem(row0, NUM_POS))

        pltpu.sync_copy(buf_v, out_hbm.at[pl.ds(row0, CHUNK)])
        return 0

    lax.fori_loop(0, NUM_CHUNKS, chunk_body, 0)


def kernel(input_tokens, token_table, position_table):
    tok = input_tokens.astype(jnp.int32).reshape(ROWS)
    out = _emb_kernel(tok, token_table, position_table)
    return out.reshape(BATCH, NUM_POS, EMBED)


# speedup vs baseline: 1.0902x; 1.0902x over previous
import functools

import jax
import jax.numpy as jnp
from jax import lax
from jax.experimental import pallas as pl
from jax.experimental.pallas import tpu as pltpu
from jax.experimental.pallas import tpu_sc as plsc

VOCAB = 49408
NUM_POS = 77
EMBED = 768
BATCH = 1024
ROWS = BATCH * NUM_POS

NUM_CORES = 2
NUM_SUBCORES = 16
NUM_WORKERS = NUM_CORES * NUM_SUBCORES
ROWS_PER_W = ROWS // NUM_WORKERS
CHUNK = 16
NUM_CHUNKS = ROWS_PER_W // CHUNK
LANES = 16
VREGS_PER_ROW = EMBED // LANES

_mesh = plsc.VectorSubcoreMesh(core_axis_name="c", subcore_axis_name="s")


@functools.partial(
    pl.kernel,
    mesh=_mesh,
    out_type=jax.ShapeDtypeStruct((ROWS, EMBED), jnp.float32),
    scratch_types=[
        pltpu.VMEM((ROWS_PER_W,), jnp.int32),
        pltpu.VMEM((NUM_POS, EMBED), jnp.float32),
        pltpu.VMEM((CHUNK, EMBED), jnp.float32),
        pltpu.VMEM((CHUNK, EMBED), jnp.float32),
        pltpu.SemaphoreType.DMA,
        pltpu.SemaphoreType.DMA,
        pltpu.SemaphoreType.DMA,
        pltpu.SemaphoreType.DMA,
        pltpu.SemaphoreType.DMA,
    ],
)
def _emb_kernel(tok_hbm, table_hbm, pos_hbm, out_hbm, idx_v, pos_v,
                buf0, buf1, g0, g1, s0, s1, sp):
    wid = lax.axis_index("s") * NUM_CORES + lax.axis_index("c")
    wrow0 = wid * ROWS_PER_W
    bufs, gsems, ssems = (buf0, buf1), (g0, g1), (s0, s1)

    def gather_desc(c, b):
        return pltpu.make_async_copy(
            table_hbm.at[idx_v.at[pl.ds(c * CHUNK, CHUNK)]], bufs[b], gsems[b])

    def store_desc(c, b):
        return pltpu.make_async_copy(
            bufs[b], out_hbm.at[pl.ds(wrow0 + c * CHUNK, CHUNK)], ssems[b])

    pos_cp = pltpu.make_async_copy(pos_hbm, pos_v, sp)
    pos_cp.start()
    pltpu.sync_copy(tok_hbm.at[pl.ds(wrow0, ROWS_PER_W)], idx_v)
    pos_cp.wait()

    gather_desc(0, 0).start()

    def pair_body(g, _):
        for b in (0, 1):
            c = 2 * g + b
            gather_desc(c, b).wait()

            @pl.when(c >= 1)
            def _():
                store_desc(c - 1, 1 - b).wait()

            @pl.when(c + 1 < NUM_CHUNKS)
            def _():
                gather_desc(c + 1, 1 - b).start()

            buf = bufs[b]

            def add_row(k, phase):
                for j in range(VREGS_PER_ROW):
                    sl = pl.ds(j * LANES, LANES)
                    buf[k, sl] = buf[k, sl] + pos_v[phase, sl]
                return jnp.where(phase == NUM_POS - 1, 0, phase + 1)

            lax.fori_loop(0, CHUNK, add_row, lax.rem(c * CHUNK, NUM_POS))

            store_desc(c, b).start()
        return 0

    lax.fori_loop(0, NUM_CHUNKS // 2, pair_body, 0)
    store_desc(NUM_CHUNKS - 1, 1).wait()


def kernel(input_tokens, token_table, position_table):
    tok = input_tokens.astype(jnp.int32).reshape(ROWS)
    out = _emb_kernel(tok, token_table, position_table)
    return out.reshape(BATCH, NUM_POS, EMBED)


# speedup vs baseline: 1.8796x; 1.7242x over previous
import functools

import jax
import jax.numpy as jnp
from jax import lax
from jax.experimental import pallas as pl
from jax.experimental.pallas import tpu as pltpu
from jax.experimental.pallas import tpu_sc as plsc

VOCAB = 49408
NUM_POS = 77
EMBED = 768
BATCH = 1024

NUM_CORES = 2
NUM_SUBCORES = 16
NUM_WORKERS = NUM_CORES * NUM_SUBCORES
HALF = EMBED // 2
ITEMS_PER_W = BATCH // (NUM_WORKERS // 2)
POS_PAD = 80
MAIN = 72
TAIL = NUM_POS - MAIN
LANES = 16
VREGS_PER_HROW = HALF // LANES

_mesh = plsc.VectorSubcoreMesh(core_axis_name="c", subcore_axis_name="s")


@functools.partial(
    pl.kernel,
    mesh=_mesh,
    out_type=jax.ShapeDtypeStruct((BATCH, NUM_POS, EMBED), jnp.float32),
    scratch_types=[
        pltpu.VMEM((ITEMS_PER_W, POS_PAD), jnp.int32),
        pltpu.VMEM((NUM_POS, HALF), jnp.float32),
        pltpu.VMEM((NUM_POS, HALF), jnp.float32),
        pltpu.VMEM((NUM_POS, HALF), jnp.float32),
        pltpu.VMEM((8, HALF), jnp.float32),
        pltpu.VMEM((8, HALF), jnp.float32),
        pltpu.SemaphoreType.DMA,
        pltpu.SemaphoreType.DMA,
        pltpu.SemaphoreType.DMA,
        pltpu.SemaphoreType.DMA,
        pltpu.SemaphoreType.DMA,
    ],
)
def _emb_kernel(tok_hbm, table_hbm, pos_hbm, out_hbm, idx_v, pos_v,
                buf0, buf1, tb0, tb1, g0, g1, s0, s1, sp):
    wid = lax.axis_index("s") * NUM_CORES + lax.axis_index("c")
    grp = wid // 2
    item0 = grp * ITEMS_PER_W
    bufs, tbufs, gsems, ssems = (buf0, buf1), (tb0, tb1), (g0, g1), (s0, s1)

    def run(off):
        def main_desc(n, b):
            return pltpu.make_async_copy(
                table_hbm.at[idx_v.at[n, pl.ds(0, MAIN)], pl.ds(off, HALF)],
                bufs[b].at[pl.ds(0, MAIN)], gsems[b])

        def tail_desc(n, b):
            return pltpu.make_async_copy(
                table_hbm.at[idx_v.at[n, pl.ds(MAIN, 8)], pl.ds(off, HALF)],
                tbufs[b], gsems[b])

        def store_desc(n, b):
            return pltpu.make_async_copy(
                bufs[b], out_hbm.at[item0 + n, :, pl.ds(off, HALF)], ssems[b])

        pos_cp = pltpu.make_async_copy(pos_hbm.at[:, pl.ds(off, HALF)],
                                       pos_v, sp)
        pos_cp.start()
        pltpu.sync_copy(tok_hbm.at[pl.ds(item0, ITEMS_PER_W)], idx_v)
        pos_cp.wait()

        main_desc(0, 0).start()
        tail_desc(0, 0).start()

        def pair_body(g, _):
            for b in (0, 1):
                n = 2 * g + b
                main_desc(n, b).wait()
                tail_desc(n, b).wait()

                @pl.when(n >= 1)
                def _():
                    store_desc(n - 1, 1 - b).wait()

                @pl.when(n + 1 < ITEMS_PER_W)
                def _():
                    main_desc(n + 1, 1 - b).start()
                    tail_desc(n + 1, 1 - b).start()

                buf, tbuf = bufs[b], tbufs[b]

                def add_row(r, _):
                    for j in range(VREGS_PER_HROW):
                        sl = pl.ds(j * LANES, LANES)
                        buf[r, sl] = buf[r, sl] + pos_v[r, sl]
                    return 0

                lax.fori_loop(0, MAIN, add_row, 0)

                for t in range(TAIL):
                    for j in range(VREGS_PER_HROW):
                        sl = pl.ds(j * LANES, LANES)
                        buf[MAIN + t, sl] = tbuf[t, sl] + pos_v[MAIN + t, sl]

                store_desc(n, b).start()
            return 0

        lax.fori_loop(0, ITEMS_PER_W // 2, pair_body, 0)
        store_desc(ITEMS_PER_W - 1, 1).wait()

    half = wid % 2

    @pl.when(half == 0)
    def _():
        run(0)

    @pl.when(half == 1)
    def _():
        run(HALF)


def kernel(input_tokens, token_table, position_table):
    tok = input_tokens.astype(jnp.int32)
    tok = jnp.pad(tok, ((0, 0), (0, POS_PAD - NUM_POS)))
    return _emb_kernel(tok, token_table, position_table)
